# baseline (device time: 63533 ns/iter reference)
import jax
import jax.numpy as jnp
from jax import lax
from jax.experimental import pallas as pl
from jax.experimental.pallas import tpu as pltpu

N_DEV = 16
SQ = 1024
SKV = 1024
DH = 128
H_LOC = 8
BLK = 64
CHUNK = SQ // N_DEV
N_SLAB = 4
SLAB = SQ // N_SLAB
SCALE = 0.08838834764831843


def _body(x_ref, wq_ref, k_any, v_any, wo_ref, out_ref,
          kf_ref, vf_ref, kbf_ref, vbf_ref, qs_ref,
          xb_ref, wqb_ref, wob_ref,
          ctx_ref, part_ref, rs_buf,
          load_sem, rs_send, rs_recv, ag_send, ag_recv):
    my = lax.axis_index("i")

    loads = []
    for h in range(H_LOC):
        sl = slice(h * DH, (h + 1) * DH)
        kc = pltpu.make_async_copy(
            k_any.at[0, :, my * H_LOC + h, :], kf_ref.at[:, sl],
            load_sem.at[h])
        vc = pltpu.make_async_copy(
            v_any.at[0, :, my * H_LOC + h, :], vf_ref.at[:, sl],
            load_sem.at[H_LOC + h])
        kc.start()
        vc.start()
        loads += [kc, vc]

    barrier = pltpu.get_barrier_semaphore()
    for j in range(1, N_DEV):
        tgt = lax.rem(my + j, N_DEV)
        pl.semaphore_signal(barrier, inc=1, device_id=(tgt,),
                            device_id_type=pl.DeviceIdType.MESH)

    xb_ref[:, :] = x_ref[:, :].astype(jnp.bfloat16)
    wqb_ref[:, :] = wq_ref[:, :].astype(jnp.bfloat16)
    qs_ref[:, :] = lax.dot_general(
        xb_ref[:, :], wqb_ref[:, :], (((1,), (0,)), ((), ())),
        preferred_element_type=jnp.float32).astype(jnp.bfloat16)
    wob_ref[:, :] = wo_ref[:, :].astype(jnp.bfloat16)

    for ld in loads:
        ld.wait()
    kbf_ref[:, :] = kf_ref[:, :].astype(jnp.bfloat16)
    vbf_ref[:, :] = vf_ref[:, :].astype(jnp.bfloat16)

    qb = lax.broadcasted_iota(jnp.int32, (SLAB, 1), 0) // BLK
    kb = lax.broadcasted_iota(jnp.int32, (1, SKV), 1) // BLK

    def compute_slab(s):
        rows = pl.ds(s * SLAB, SLAB)
        qblk = qb + s * (SLAB // BLK)
        keep = (kb == qblk) | (kb == 0) | (lax.rem(kb + qblk, 3) == 0)
        bias = jnp.where(keep, 0.0, -1e9)
        for h in range(H_LOC):
            sl = slice(h * DH, (h + 1) * DH)
            scores = lax.dot_general(
                qs_ref[rows, sl], kbf_ref[:, sl], (((1,), (1,)), ((), ())),
                preferred_element_type=jnp.float32)
            e = jnp.exp(scores * SCALE + bias)
            d = jnp.sum(e, axis=1, keepdims=True)
            ctx = lax.dot_general(
                e.astype(jnp.bfloat16), vbf_ref[:, sl],
                (((1,), (0,)), ((), ())),
                preferred_element_type=jnp.float32)
            ctx_ref[:, sl] = (ctx / d).astype(jnp.bfloat16)
        part_ref[rows, :] = lax.dot_general(
            ctx_ref[:, :], wob_ref[:, :], (((1,), (0,)), ((), ())),
            preferred_element_type=jnp.float32).astype(jnp.bfloat16)

    rs_rdmas = []

    def send_chunk(c):
        slot = lax.rem(c - my + N_DEV, N_DEV) - 1
        rdma = pltpu.make_async_remote_copy(
            src_ref=part_ref.at[pl.ds(c * CHUNK, CHUNK), :],
            dst_ref=rs_buf.at[slot],
            send_sem=rs_send.at[slot],
            recv_sem=rs_recv.at[slot],
            device_id=(c,),
            device_id_type=pl.DeviceIdType.MESH,
        )
        rdma.start()
        rs_rdmas.append(rdma)

    mygrp = my // N_SLAB
    mypos = lax.rem(my, N_SLAB)
    for t in range(1, N_SLAB):
        s = lax.rem(mygrp + t, N_SLAB)
        compute_slab(s)
        if t == 1:
            pl.semaphore_wait(barrier, N_DEV - 1)
        for q in range(N_SLAB):
            send_chunk(s * N_SLAB + q)
    compute_slab(mygrp)
    for r in range(1, N_SLAB):
        send_chunk(mygrp * N_SLAB + lax.rem(mypos + r, N_SLAB))

    acc = part_ref[pl.ds(my * CHUNK, CHUNK), :].astype(jnp.float32)
    for k in range(N_DEV - 1):
        recv = pltpu.make_async_remote_copy(
            src_ref=part_ref.at[pl.ds(my * CHUNK, CHUNK), :],
            dst_ref=rs_buf.at[k],
            send_sem=rs_send.at[k],
            recv_sem=rs_recv.at[k],
            device_id=(my,),
            device_id_type=pl.DeviceIdType.MESH,
        )
        recv.wait_recv()
        acc = acc + rs_buf[k, :, :].astype(jnp.float32)
    out_ref[pl.ds(my * CHUNK, CHUNK), :] = acc.astype(jnp.bfloat16)

    ag_rdmas = []
    for j in range(1, N_DEV):
        tgt = lax.rem(my + j, N_DEV)
        rdma = pltpu.make_async_remote_copy(
            src_ref=out_ref.at[pl.ds(my * CHUNK, CHUNK), :],
            dst_ref=out_ref.at[pl.ds(my * CHUNK, CHUNK), :],
            send_sem=ag_send.at[j - 1],
            recv_sem=ag_recv.at[j - 1],
            device_id=(tgt,),
            device_id_type=pl.DeviceIdType.MESH,
        )
        rdma.start()
        ag_rdmas.append(rdma)

    for k in range(N_DEV - 1):
        src_dev = lax.rem(my - k - 1 + N_DEV, N_DEV)
        recv = pltpu.make_async_remote_copy(
            src_ref=out_ref.at[pl.ds(my * CHUNK, CHUNK), :],
            dst_ref=out_ref.at[pl.ds(src_dev * CHUNK, CHUNK), :],
            send_sem=ag_send.at[k],
            recv_sem=ag_recv.at[k],
            device_id=(my,),
            device_id_type=pl.DeviceIdType.MESH,
        )
        recv.wait_recv()

    for rdma in rs_rdmas:
        rdma.wait_send()
    for rdma in ag_rdmas:
        rdma.wait_send()


def kernel(x, Wq, K_ext, V_ext, Wo):
    out = pl.pallas_call(
        _body,
        out_shape=jax.ShapeDtypeStruct((SQ, 1024), jnp.bfloat16),
        in_specs=[
            pl.BlockSpec(memory_space=pltpu.MemorySpace.VMEM),
            pl.BlockSpec(memory_space=pltpu.MemorySpace.VMEM),
            pl.BlockSpec(memory_space=pltpu.MemorySpace.HBM),
            pl.BlockSpec(memory_space=pltpu.MemorySpace.HBM),
            pl.BlockSpec(memory_space=pltpu.MemorySpace.VMEM),
        ],
        out_specs=pl.BlockSpec(memory_space=pltpu.MemorySpace.VMEM),
        scratch_shapes=[
            pltpu.VMEM((SKV, H_LOC * DH), jnp.float32),
            pltpu.VMEM((SKV, H_LOC * DH), jnp.float32),
            pltpu.VMEM((SKV, H_LOC * DH), jnp.bfloat16),
            pltpu.VMEM((SKV, H_LOC * DH), jnp.bfloat16),
            pltpu.VMEM((SQ, H_LOC * DH), jnp.bfloat16),
            pltpu.VMEM((SQ, 1024), jnp.bfloat16),
            pltpu.VMEM((1024, 1024), jnp.bfloat16),
            pltpu.VMEM((1024, 1024), jnp.bfloat16),
            pltpu.VMEM((SLAB, H_LOC * DH), jnp.bfloat16),
            pltpu.VMEM((SQ, 1024), jnp.bfloat16),
            pltpu.VMEM((N_DEV - 1, CHUNK, 1024), jnp.bfloat16),
            pltpu.SemaphoreType.DMA((2 * H_LOC,)),
            pltpu.SemaphoreType.DMA((N_DEV - 1,)),
            pltpu.SemaphoreType.DMA((N_DEV - 1,)),
            pltpu.SemaphoreType.DMA((N_DEV - 1,)),
            pltpu.SemaphoreType.DMA((N_DEV - 1,)),
        ],
        compiler_params=pltpu.CompilerParams(collective_id=0),
    )(x.reshape(SQ, 1024), Wq, K_ext, V_ext, Wo)
    return out.reshape(1, SQ, 1024)


# device time: 61657 ns/iter; 1.0304x vs baseline; 1.0304x over previous
import jax
import jax.numpy as jnp
from jax import lax
from jax.experimental import pallas as pl
from jax.experimental.pallas import tpu as pltpu

N_DEV = 16
SQ = 1024
SKV = 1024
DH = 128
H_LOC = 8
BLK = 64
CHUNK = SQ // N_DEV
N_SLAB = 4
SLAB = SQ // N_SLAB
SCALE = 0.08838834764831843


def _body(x_ref, wq_ref, k_any, v_any, wo_ref, out_ref,
          kf_ref, vf_ref, kbf_ref, vbf_ref, qs_ref,
          xb_ref, wqb_ref, wob_ref,
          ctx_ref, part_ref, rs_buf,
          load_sem, rs_send, rs_recv, ag_send, ag_recv):
    my = lax.axis_index("i")

    loads = []
    for h in range(H_LOC):
        sl = slice(h * DH, (h + 1) * DH)
        kc = pltpu.make_async_copy(
            k_any.at[0, :, my * H_LOC + h, :], kf_ref.at[:, sl],
            load_sem.at[h])
        vc = pltpu.make_async_copy(
            v_any.at[0, :, my * H_LOC + h, :], vf_ref.at[:, sl],
            load_sem.at[H_LOC + h])
        kc.start()
        vc.start()
        loads += [kc, vc]

    barrier = pltpu.get_barrier_semaphore()
    for j in range(1, N_DEV):
        tgt = lax.rem(my + j, N_DEV)
        pl.semaphore_signal(barrier, inc=1, device_id=(tgt,),
                            device_id_type=pl.DeviceIdType.MESH)

    xb_ref[:, :] = x_ref[:, :].astype(jnp.bfloat16)
    wqb_ref[:, :] = wq_ref[:, :].astype(jnp.bfloat16)
    qs_ref[:, :] = lax.dot_general(
        xb_ref[:, :], wqb_ref[:, :], (((1,), (0,)), ((), ())),
        preferred_element_type=jnp.float32).astype(jnp.bfloat16)
    wob_ref[:, :] = wo_ref[:, :].astype(jnp.bfloat16)

    for ld in loads:
        ld.wait()
    kbf_ref[:, :] = kf_ref[:, :].astype(jnp.bfloat16)
    vbf_ref[:, :] = vf_ref[:, :].astype(jnp.bfloat16)

    qb = lax.broadcasted_iota(jnp.int32, (SLAB, 1), 0) // BLK
    kb = lax.broadcasted_iota(jnp.int32, (1, SKV), 1) // BLK

    def compute_slab(s):
        rows = pl.ds(s * SLAB, SLAB)
        qblk = qb + s * (SLAB // BLK)
        keep = (kb == qblk) | (kb == 0) | (lax.rem(kb + qblk, 3) == 0)
        bias = jnp.where(keep, 0.0, -1e9)
        for h in range(H_LOC):
            sl = slice(h * DH, (h + 1) * DH)
            scores = lax.dot_general(
                qs_ref[rows, sl], kbf_ref[:, sl], (((1,), (1,)), ((), ())),
                preferred_element_type=jnp.float32)
            e = jnp.exp(scores * SCALE + bias)
            d = jnp.sum(e, axis=1, keepdims=True)
            ctx = lax.dot_general(
                e.astype(jnp.bfloat16), vbf_ref[:, sl],
                (((1,), (0,)), ((), ())),
                preferred_element_type=jnp.float32)
            ctx_ref[:, sl] = (ctx / d).astype(jnp.bfloat16)
        part_ref[rows, :] = lax.dot_general(
            ctx_ref[:, :], wob_ref[:, :], (((1,), (0,)), ((), ())),
            preferred_element_type=jnp.float32).astype(jnp.bfloat16)

    rs_rdmas = []

    def send_chunk_to(c, owner):
        slot = lax.rem(owner - my + N_DEV, N_DEV) - 1
        rdma = pltpu.make_async_remote_copy(
            src_ref=part_ref.at[pl.ds(c * CHUNK, CHUNK), :],
            dst_ref=rs_buf.at[slot],
            send_sem=rs_send.at[slot],
            recv_sem=rs_recv.at[slot],
            device_id=(owner,),
            device_id_type=pl.DeviceIdType.MESH,
        )
        rdma.start()
        rs_rdmas.append(rdma)

    mygrp = my // N_SLAB
    mypos = lax.rem(my, N_SLAB)
    own_chunk = N_SLAB * mypos + mygrp

    def send_slab_chunk(s, q):
        send_chunk_to(s * N_SLAB + q, N_SLAB * q + s)

    for t in range(1, N_SLAB):
        s = lax.rem(mypos + t, N_SLAB)
        compute_slab(s)
        if t == 1:
            pl.semaphore_wait(barrier, N_DEV - 1)
        for q in range(N_SLAB):
            send_slab_chunk(s, q)
    compute_slab(mypos)
    for r in range(1, N_SLAB):
        send_slab_chunk(mypos, lax.rem(mygrp + r, N_SLAB))

    acc = part_ref[pl.ds(own_chunk * CHUNK, CHUNK), :].astype(jnp.float32)
    for k in range(N_DEV - 1):
        recv = pltpu.make_async_remote_copy(
            src_ref=part_ref.at[pl.ds(own_chunk * CHUNK, CHUNK), :],
            dst_ref=rs_buf.at[k],
            send_sem=rs_send.at[k],
            recv_sem=rs_recv.at[k],
            device_id=(my,),
            device_id_type=pl.DeviceIdType.MESH,
        )
        recv.wait_recv()
        acc = acc + rs_buf[k, :, :].astype(jnp.float32)
    out_ref[pl.ds(own_chunk * CHUNK, CHUNK), :] = acc.astype(jnp.bfloat16)

    ag_rdmas = []
    for j in range(1, N_DEV):
        tgt = lax.rem(my + j, N_DEV)
        rdma = pltpu.make_async_remote_copy(
            src_ref=out_ref.at[pl.ds(own_chunk * CHUNK, CHUNK), :],
            dst_ref=out_ref.at[pl.ds(own_chunk * CHUNK, CHUNK), :],
            send_sem=ag_send.at[j - 1],
            recv_sem=ag_recv.at[j - 1],
            device_id=(tgt,),
            device_id_type=pl.DeviceIdType.MESH,
        )
        rdma.start()
        ag_rdmas.append(rdma)

    for k in range(N_DEV - 1):
        src_dev = lax.rem(my - k - 1 + N_DEV, N_DEV)
        src_chunk = N_SLAB * lax.rem(src_dev, N_SLAB) + src_dev // N_SLAB
        recv = pltpu.make_async_remote_copy(
            src_ref=out_ref.at[pl.ds(own_chunk * CHUNK, CHUNK), :],
            dst_ref=out_ref.at[pl.ds(src_chunk * CHUNK, CHUNK), :],
            send_sem=ag_send.at[k],
            recv_sem=ag_recv.at[k],
            device_id=(my,),
            device_id_type=pl.DeviceIdType.MESH,
        )
        recv.wait_recv()

    for rdma in rs_rdmas:
        rdma.wait_send()
    for rdma in ag_rdmas:
        rdma.wait_send()


def kernel(x, Wq, K_ext, V_ext, Wo):
    out = pl.pallas_call(
        _body,
        out_shape=jax.ShapeDtypeStruct((SQ, 1024), jnp.bfloat16),
        in_specs=[
            pl.BlockSpec(memory_space=pltpu.MemorySpace.VMEM),
            pl.BlockSpec(memory_space=pltpu.MemorySpace.VMEM),
            pl.BlockSpec(memory_space=pltpu.MemorySpace.HBM),
            pl.BlockSpec(memory_space=pltpu.MemorySpace.HBM),
            pl.BlockSpec(memory_space=pltpu.MemorySpace.VMEM),
        ],
        out_specs=pl.BlockSpec(memory_space=pltpu.MemorySpace.VMEM),
        scratch_shapes=[
            pltpu.VMEM((SKV, H_LOC * DH), jnp.float32),
            pltpu.VMEM((SKV, H_LOC * DH), jnp.float32),
            pltpu.VMEM((SKV, H_LOC * DH), jnp.bfloat16),
            pltpu.VMEM((SKV, H_LOC * DH), jnp.bfloat16),
            pltpu.VMEM((SQ, H_LOC * DH), jnp.bfloat16),
            pltpu.VMEM((SQ, 1024), jnp.bfloat16),
            pltpu.VMEM((1024, 1024), jnp.bfloat16),
            pltpu.VMEM((1024, 1024), jnp.bfloat16),
            pltpu.VMEM((SLAB, H_LOC * DH), jnp.bfloat16),
            pltpu.VMEM((SQ, 1024), jnp.bfloat16),
            pltpu.VMEM((N_DEV - 1, CHUNK, 1024), jnp.bfloat16),
            pltpu.SemaphoreType.DMA((2 * H_LOC,)),
            pltpu.SemaphoreType.DMA((N_DEV - 1,)),
            pltpu.SemaphoreType.DMA((N_DEV - 1,)),
            pltpu.SemaphoreType.DMA((N_DEV - 1,)),
            pltpu.SemaphoreType.DMA((N_DEV - 1,)),
        ],
        compiler_params=pltpu.CompilerParams(collective_id=0),
    )(x.reshape(SQ, 1024), Wq, K_ext, V_ext, Wo)
    return out.reshape(1, SQ, 1024)
